# baseline (speedup 1.0000x reference)
import functools

import jax
import jax.numpy as jnp
from jax import lax
from jax.experimental import pallas as pl
from jax.experimental.pallas import tpu as pltpu
from jax.experimental.pallas import tpu_sc as plsc

F32 = jnp.float32
NSUB = 16
NCORE = 2
BATCH = 128
DEGW = 16


def _sc_degree(dstp, zdeg, nacc, eps):
  nb = eps // BATCH
  nw = NCORE * NSUB
  mesh = plsc.VectorSubcoreMesh(core_axis_name="c", subcore_axis_name="s")

  @functools.partial(
      pl.kernel,
      out_type=jax.ShapeDtypeStruct((nw, nacc), F32),
      mesh=mesh,
      scratch_types=[
          pltpu.VMEM((BATCH,), jnp.int32),
          pltpu.VMEM((nacc,), F32),
      ],
      compiler_params=pltpu.CompilerParams(needs_layout_passes=False),
  )
  def deg_kernel(dst_hbm, z_hbm, out_hbm, idxb, hist):
    c = lax.axis_index("c")
    s = lax.axis_index("s")
    w = c * NSUB + s
    pltpu.sync_copy(z_hbm, hist)
    base0 = w * eps

    @pl.loop(0, nb)
    def _(i):
      pltpu.sync_copy(dst_hbm.at[pl.ds(base0 + i * BATCH, BATCH)], idxb)
      for j in range(BATCH // 16):
        d16 = idxb[pl.ds(j * 16, 16)]
        cnt, last = plsc.scan_count(d16)
        old = plsc.load_gather(hist, [d16], mask=last)
        plsc.store_scatter(hist, [d16], old + cnt.astype(F32), mask=last)

    pltpu.sync_copy(hist, out_hbm.at[w])

  return deg_kernel(dstp, zdeg)


def _sc_prop(y, srcp, dstp, zhbm, nacc, eps, d):
  nb = eps // BATCH
  rows_sub = nacc // NSUB
  mesh = plsc.VectorSubcoreMesh(core_axis_name="c", subcore_axis_name="s")

  @functools.partial(
      pl.kernel,
      out_type=jax.ShapeDtypeStruct((NCORE, nacc, d), F32),
      mesh=mesh,
      scratch_types=[
          pltpu.VMEM((BATCH,), jnp.int32),
          pltpu.VMEM((BATCH,), jnp.int32),
          pltpu.VMEM((BATCH,), jnp.int32),
          pltpu.VMEM((BATCH,), jnp.int32),
          pltpu.VMEM((BATCH, d), F32),
          pltpu.VMEM((BATCH, d), F32),
          pltpu.VMEM_SHARED((nacc, d), F32),
          pltpu.SemaphoreType.DMA,
          pltpu.SemaphoreType.DMA,
      ],
  )
  def prop_kernel(y_hbm, src_hbm, dst_hbm, z_hbm, out_hbm,
                  srcb0, srcb1, dstb0, dstb1, rows0, rows1, acc, sem0, sem1):
    c = lax.axis_index("c")
    s = lax.axis_index("s")
    w = c * NSUB + s
    pltpu.sync_copy(z_hbm.at[pl.ds(s * rows_sub, rows_sub)],
                    acc.at[pl.ds(s * rows_sub, rows_sub)])
    plsc.subcore_barrier()
    base0 = w * eps

    def load_and_gather(i, srcb, dstb, rows, sem):
      b0 = base0 + i * BATCH
      pltpu.sync_copy(src_hbm.at[pl.ds(b0, BATCH)], srcb)
      pltpu.sync_copy(dst_hbm.at[pl.ds(b0, BATCH)], dstb)
      pltpu.async_copy(y_hbm.at[srcb], rows, sem)

    load_and_gather(0, srcb0, dstb0, rows0, sem0)
    load_and_gather(1, srcb1, dstb1, rows1, sem1)

    @pl.loop(0, nb // 2)
    def _(k):
      i = 2 * k
      pltpu.make_async_copy(y_hbm.at[srcb0], rows0, sem0).wait()
      pltpu.sync_copy(rows0, acc.at[dstb0], add=True)

      @pl.when(i + 2 < nb)
      def _():
        load_and_gather(i + 2, srcb0, dstb0, rows0, sem0)

      pltpu.make_async_copy(y_hbm.at[srcb1], rows1, sem1).wait()
      pltpu.sync_copy(rows1, acc.at[dstb1], add=True)

      @pl.when(i + 3 < nb)
      def _():
        load_and_gather(i + 3, srcb1, dstb1, rows1, sem1)

    plsc.subcore_barrier()
    pltpu.sync_copy(acc.at[pl.ds(s * rows_sub, rows_sub)],
                    out_hbm.at[c, pl.ds(s * rows_sub, rows_sub)])

  return prop_kernel(y, srcp, dstp, zhbm)


def _tc_layer1(x, w1, degp, n, nfeat, nhid, rb):
  grid = n // rb
  nw = degp.shape[1]

  def body(x_ref, w_ref, dp_ref, y_ref, dinv_ref):
    ones = jnp.ones((nw, 1), F32)
    deg = jnp.dot(dp_ref[...], ones, preferred_element_type=F32) + 1.0
    dinv = lax.rsqrt(jnp.maximum(deg, 1.0))
    h = jnp.dot(x_ref[...], w_ref[...], preferred_element_type=F32)
    y_ref[...] = dinv * h
    dinv_ref[...] = dinv

  return pl.pallas_call(
      body,
      grid=(grid,),
      in_specs=[
          pl.BlockSpec((rb, nfeat), lambda i: (i, 0)),
          pl.BlockSpec((nfeat, nhid), lambda i: (0, 0)),
          pl.BlockSpec((rb, nw), lambda i: (i, 0)),
      ],
      out_specs=[
          pl.BlockSpec((rb, nhid), lambda i: (i, 0)),
          pl.BlockSpec((rb, 1), lambda i: (i, 0)),
      ],
      out_shape=[
          jax.ShapeDtypeStruct((n, nhid), F32),
          jax.ShapeDtypeStruct((n, 1), F32),
      ],
  )(x, w1, degp)


def _tc_layer2(p0, p1, y1, dinv, b1r, w2p, n, nhid, ncp, rb):
  grid = n // rb

  def body(p0_ref, p1_ref, y1_ref, dinv_ref, b1_ref, w_ref, y2_ref):
    a = p0_ref[...] + p1_ref[...] + y1_ref[...]
    t = jnp.maximum(dinv_ref[...] * a + b1_ref[...], 0.0)
    h2 = jnp.dot(t, w_ref[...], preferred_element_type=F32)
    y2_ref[...] = dinv_ref[...] * h2

  return pl.pallas_call(
      body,
      grid=(grid,),
      in_specs=[
          pl.BlockSpec((rb, nhid), lambda i: (i, 0)),
          pl.BlockSpec((rb, nhid), lambda i: (i, 0)),
          pl.BlockSpec((rb, nhid), lambda i: (i, 0)),
          pl.BlockSpec((rb, 1), lambda i: (i, 0)),
          pl.BlockSpec((1, nhid), lambda i: (0, 0)),
          pl.BlockSpec((nhid, ncp), lambda i: (0, 0)),
      ],
      out_specs=pl.BlockSpec((rb, ncp), lambda i: (i, 0)),
      out_shape=jax.ShapeDtypeStruct((n, ncp), F32),
  )(p0, p1, y1, dinv, b1r, w2p)


def _tc_out(p0, p1, y2, dinv, b2r, n, ncp, ncls, rb):
  grid = n // rb

  def body(p0_ref, p1_ref, y2_ref, dinv_ref, b2_ref, out_ref):
    v = dinv_ref[...] * (p0_ref[...] + p1_ref[...] + y2_ref[...]) + b2_ref[...]
    out_ref[...] = v[:, :ncls]

  return pl.pallas_call(
      body,
      grid=(grid,),
      in_specs=[
          pl.BlockSpec((rb, ncp), lambda i: (i, 0)),
          pl.BlockSpec((rb, ncp), lambda i: (i, 0)),
          pl.BlockSpec((rb, ncp), lambda i: (i, 0)),
          pl.BlockSpec((rb, 1), lambda i: (i, 0)),
          pl.BlockSpec((1, ncp), lambda i: (0, 0)),
      ],
      out_specs=pl.BlockSpec((rb, ncls), lambda i: (i, 0)),
      out_shape=jax.ShapeDtypeStruct((n, ncls), F32),
  )(p0, p1, y2, dinv, b2r)


def kernel(x, edge_index, W1, b1, W2, b2):
  n, nfeat = x.shape
  nhid = W1.shape[1]
  ncls = W2.shape[1]
  e = edge_index.shape[1]
  nw = NCORE * NSUB

  eps = -(-e // (nw * 2 * BATCH)) * 2 * BATCH
  etot = eps * nw
  nacc = -(-(n + 1) // 128) * 128
  ncp = 128
  rb = 1000

  ei = edge_index.astype(jnp.int32)
  pad = etot - e
  srcp = jnp.concatenate([ei[0], jnp.zeros((pad,), jnp.int32)])
  dstp = jnp.concatenate([ei[1], jnp.full((pad,), n, jnp.int32)])

  zdeg = jnp.zeros((nacc,), F32)
  z1 = jnp.zeros((nacc, nhid), F32)
  z2 = jnp.zeros((nacc, ncp), F32)
  w2p = jnp.pad(W2, ((0, 0), (0, ncp - ncls)))
  b1r = b1.reshape(1, nhid)
  b2r = jnp.pad(b2, (0, ncp - ncls)).reshape(1, ncp)

  degp = _sc_degree(dstp, zdeg, nacc, eps).T
  y1, dinv = _tc_layer1(x, W1, degp, n, nfeat, nhid, rb)
  p1 = _sc_prop(y1, srcp, dstp, z1, nacc, eps, nhid)
  y2 = _tc_layer2(p1[0], p1[1], y1, dinv, b1r, w2p, n, nhid, ncp, rb)
  p2 = _sc_prop(y2, srcp, dstp, z2, nacc, eps, ncp)
  return _tc_out(p2[0], p2[1], y2, dinv, b2r, n, ncp, ncls, rb)


# speedup vs baseline: 2.6299x; 2.6299x over previous
import functools

import jax
import jax.numpy as jnp
from jax import lax
from jax.experimental import pallas as pl
from jax.experimental.pallas import tpu as pltpu
from jax.experimental.pallas import tpu_sc as plsc

F32 = jnp.float32
NSUB = 16
NCORE = 2
BATCH = 128
DEGW = 16


def _sc_degree(dstp, zdeg, nacc, eps):
  nb = eps // BATCH
  nw = NCORE * NSUB
  mesh = plsc.VectorSubcoreMesh(core_axis_name="c", subcore_axis_name="s")

  @functools.partial(
      pl.kernel,
      out_type=jax.ShapeDtypeStruct((nw, nacc), F32),
      mesh=mesh,
      scratch_types=[
          pltpu.VMEM((BATCH,), jnp.int32),
          pltpu.VMEM((nacc,), F32),
      ],
      compiler_params=pltpu.CompilerParams(needs_layout_passes=False),
  )
  def deg_kernel(dst_hbm, z_hbm, out_hbm, idxb, hist):
    c = lax.axis_index("c")
    s = lax.axis_index("s")
    w = c * NSUB + s
    pltpu.sync_copy(z_hbm, hist)
    base0 = w * eps

    @pl.loop(0, nb)
    def _(i):
      pltpu.sync_copy(dst_hbm.at[pl.ds(base0 + i * BATCH, BATCH)], idxb)
      for j in range(BATCH // 16):
        d16 = idxb[pl.ds(j * 16, 16)]
        cnt, last = plsc.scan_count(d16)
        old = plsc.load_gather(hist, [d16], mask=last)
        plsc.store_scatter(hist, [d16], old + cnt.astype(F32), mask=last)

    pltpu.sync_copy(hist, out_hbm.at[w])

  return deg_kernel(dstp, zdeg)


def _sc_prop(y, srcp, dstp, zhbm, nacc, eps, d):
  nb = eps // BATCH
  rows_sub = nacc // NSUB
  mesh = plsc.VectorSubcoreMesh(core_axis_name="c", subcore_axis_name="s")

  @functools.partial(
      pl.kernel,
      out_type=jax.ShapeDtypeStruct((NCORE, nacc, d), F32),
      mesh=mesh,
      scratch_types=[
          pltpu.VMEM((BATCH,), jnp.int32),
          pltpu.VMEM((BATCH,), jnp.int32),
          pltpu.VMEM((BATCH,), jnp.int32),
          pltpu.VMEM((BATCH,), jnp.int32),
          pltpu.VMEM((BATCH, d), F32),
          pltpu.VMEM((BATCH, d), F32),
          pltpu.VMEM_SHARED((nacc, d), F32),
          pltpu.SemaphoreType.DMA,
          pltpu.SemaphoreType.DMA,
      ],
  )
  def prop_kernel(y_hbm, src_hbm, dst_hbm, z_hbm, out_hbm,
                  srcb0, srcb1, dstb0, dstb1, rows0, rows1, acc, sem0, sem1):
    c = lax.axis_index("c")
    s = lax.axis_index("s")
    w = c * NSUB + s
    pltpu.sync_copy(z_hbm.at[pl.ds(s * rows_sub, rows_sub)],
                    acc.at[pl.ds(s * rows_sub, rows_sub)])
    plsc.subcore_barrier()
    base0 = w * eps

    def load_and_gather(i, srcb, dstb, rows, sem):
      b0 = base0 + i * BATCH
      pltpu.sync_copy(src_hbm.at[pl.ds(b0, BATCH)], srcb)
      pltpu.sync_copy(dst_hbm.at[pl.ds(b0, BATCH)], dstb)
      pltpu.async_copy(y_hbm.at[srcb], rows, sem)

    load_and_gather(0, srcb0, dstb0, rows0, sem0)
    load_and_gather(1, srcb1, dstb1, rows1, sem1)

    @pl.loop(0, nb // 2)
    def _(k):
      i = 2 * k
      pltpu.make_async_copy(y_hbm.at[srcb0], rows0, sem0).wait()
      pltpu.sync_copy(rows0, acc.at[dstb0], add=True)

      @pl.when(i + 2 < nb)
      def _():
        load_and_gather(i + 2, srcb0, dstb0, rows0, sem0)

      pltpu.make_async_copy(y_hbm.at[srcb1], rows1, sem1).wait()
      pltpu.sync_copy(rows1, acc.at[dstb1], add=True)

      @pl.when(i + 3 < nb)
      def _():
        load_and_gather(i + 3, srcb1, dstb1, rows1, sem1)

    plsc.subcore_barrier()
    pltpu.sync_copy(acc.at[pl.ds(s * rows_sub, rows_sub)],
                    out_hbm.at[c, pl.ds(s * rows_sub, rows_sub)])

  return prop_kernel(y, srcp, dstp, zhbm)


def _tc_layer1(x, w1, degp, n, nfeat, nhid, rb):
  grid = n // rb
  nw = degp.shape[1]

  def body(x_ref, w_ref, dp_ref, y_ref, dinv_ref):
    ones = jnp.ones((nw, 1), F32)
    deg = jnp.dot(dp_ref[...], ones, preferred_element_type=F32) + 1.0
    dinv = lax.rsqrt(jnp.maximum(deg, 1.0))
    h = jnp.dot(x_ref[...], w_ref[...], preferred_element_type=F32)
    y_ref[...] = dinv * h
    dinv_ref[...] = dinv

  return pl.pallas_call(
      body,
      grid=(grid,),
      in_specs=[
          pl.BlockSpec((rb, nfeat), lambda i: (i, 0)),
          pl.BlockSpec((nfeat, nhid), lambda i: (0, 0)),
          pl.BlockSpec((rb, nw), lambda i: (i, 0)),
      ],
      out_specs=[
          pl.BlockSpec((rb, nhid), lambda i: (i, 0)),
          pl.BlockSpec((rb, 1), lambda i: (i, 0)),
      ],
      out_shape=[
          jax.ShapeDtypeStruct((n, nhid), F32),
          jax.ShapeDtypeStruct((n, 1), F32),
      ],
  )(x, w1, degp)


def _tc_layer2(p0, p1, y1, dinv, b1r, w2p, n, nhid, ncp, rb):
  grid = n // rb

  def body(p0_ref, p1_ref, y1_ref, dinv_ref, b1_ref, w_ref, y2_ref):
    a = p0_ref[...] + p1_ref[...] + y1_ref[...]
    t = jnp.maximum(dinv_ref[...] * a + b1_ref[...], 0.0)
    h2 = jnp.dot(t, w_ref[...], preferred_element_type=F32)
    y2_ref[...] = dinv_ref[...] * h2

  return pl.pallas_call(
      body,
      grid=(grid,),
      in_specs=[
          pl.BlockSpec((rb, nhid), lambda i: (i, 0)),
          pl.BlockSpec((rb, nhid), lambda i: (i, 0)),
          pl.BlockSpec((rb, nhid), lambda i: (i, 0)),
          pl.BlockSpec((rb, 1), lambda i: (i, 0)),
          pl.BlockSpec((1, nhid), lambda i: (0, 0)),
          pl.BlockSpec((nhid, ncp), lambda i: (0, 0)),
      ],
      out_specs=pl.BlockSpec((rb, ncp), lambda i: (i, 0)),
      out_shape=jax.ShapeDtypeStruct((n, ncp), F32),
  )(p0, p1, y1, dinv, b1r, w2p)


def _tc_out(p0, p1, y2, dinv, b2r, n, ncp, ncls, rb):
  grid = n // rb

  def body(p0_ref, p1_ref, y2_ref, dinv_ref, b2_ref, out_ref):
    v = dinv_ref[...] * (p0_ref[...] + p1_ref[...] + y2_ref[...]) + b2_ref[...]
    out_ref[...] = v[:, :ncls]

  return pl.pallas_call(
      body,
      grid=(grid,),
      in_specs=[
          pl.BlockSpec((rb, ncp), lambda i: (i, 0)),
          pl.BlockSpec((rb, ncp), lambda i: (i, 0)),
          pl.BlockSpec((rb, ncp), lambda i: (i, 0)),
          pl.BlockSpec((rb, 1), lambda i: (i, 0)),
          pl.BlockSpec((1, ncp), lambda i: (0, 0)),
      ],
      out_specs=pl.BlockSpec((rb, ncls), lambda i: (i, 0)),
      out_shape=jax.ShapeDtypeStruct((n, ncls), F32),
  )(p0, p1, y2, dinv, b2r)


def kernel(x, edge_index, W1, b1, W2, b2):
  n, nfeat = x.shape
  nhid = W1.shape[1]
  ncls = W2.shape[1]
  e = edge_index.shape[1]
  nw = NCORE * NSUB

  eps = -(-e // (nw * 2 * BATCH)) * 2 * BATCH
  etot = eps * nw
  nacc = -(-(n + BATCH) // 128) * 128
  ncp = 128
  rb = 1000

  ei = edge_index.astype(jnp.int32)
  pad = etot - e
  spread = jnp.arange(pad, dtype=jnp.int32) % BATCH
  srcp = jnp.concatenate([ei[0], spread])
  dstp = jnp.concatenate([ei[1], n + spread])

  zdeg = jnp.zeros((nacc,), F32)
  z1 = jnp.zeros((nacc, nhid), F32)
  z2 = jnp.zeros((nacc, ncp), F32)
  w2p = jnp.pad(W2, ((0, 0), (0, ncp - ncls)))
  b1r = b1.reshape(1, nhid)
  b2r = jnp.pad(b2, (0, ncp - ncls)).reshape(1, ncp)

  degp = _sc_degree(dstp, zdeg, nacc, eps).T
  y1, dinv = _tc_layer1(x, W1, degp, n, nfeat, nhid, rb)
  p1 = _sc_prop(y1, srcp, dstp, z1, nacc, eps, nhid)
  y2 = _tc_layer2(p1[0], p1[1], y1, dinv, b1r, w2p, n, nhid, ncp, rb)
  p2 = _sc_prop(y2, srcp, dstp, z2, nacc, eps, ncp)
  return _tc_out(p2[0], p2[1], y2, dinv, b2r, n, ncp, ncls, rb)
